# Initial kernel scaffold; baseline (speedup 1.0000x reference)
import jax
import jax.numpy as jnp
from jax.experimental import pallas as pl


def kernel(x, edge_index, batch, W1, b1, W2, b2, W3, b3, Wlin, blin):
    raise NotImplementedError("write your pallas kernel here")


# speedup vs baseline: 25.5175x; 25.5175x over previous
import functools

import jax
import jax.numpy as jnp
from jax import lax
from jax.experimental import pallas as pl
from jax.experimental.pallas import tpu as pltpu
from jax.experimental.pallas import tpu_sc as plsc

N = 50000
F_IN = 7
H = 64
C = 2
G = 512
E = 800000

NC = 2
NS = 16
CH = 128
EP = 851968
NCHUNK = EP // CH
ROWS_PER_SC_TILE = NCHUNK // NS
ROWS_PER_WORKER = NCHUNK // (NC * NS)
BLK = 16
NP = 51200
STRIPE = NP // NS
BP = 65536
GP = 640

_mesh = plsc.VectorSubcoreMesh(core_axis_name="c", subcore_axis_name="s")
_f32 = jnp.float32
_SC_PARAMS = pltpu.CompilerParams(needs_layout_passes=False,
                                  use_tc_tiling_on_sc=False)


@functools.partial(
    pl.kernel,
    out_type=(jax.ShapeDtypeStruct((NC * NS * N,), _f32),
              jax.ShapeDtypeStruct((NC * NS * G,), _f32)),
    mesh=_mesh,
    compiler_params=_SC_PARAMS,
    scratch_types=[
        pltpu.VMEM((N,), _f32),
        pltpu.VMEM((G,), _f32),
        pltpu.VMEM((BLK, CH), jnp.int32),
    ],
)
def _deg_counts(dst_hbm, batch_hbm, degp_hbm, cntp_hbm, deg_v, cnt_v, blk_v):
    c = lax.axis_index("c")
    s = lax.axis_index("s")
    w = s * NC + c
    zero16 = jnp.zeros((16,), _f32)
    one16 = jnp.ones((16,), _f32)

    def _z(i, _):
        deg_v[pl.ds(i * 16, 16)] = zero16
        return 0

    lax.fori_loop(0, N // 16, _z, 0)
    for i in range(G // 16):
        cnt_v[pl.ds(i * 16, 16)] = zero16

    def _deg_blk(b, _):
        pltpu.sync_copy(dst_hbm.at[pl.ds(w * ROWS_PER_WORKER + b * BLK, BLK)],
                        blk_v)
        for r in range(BLK):
            for k in range(CH // 16):
                v = blk_v[r, pl.ds(k * 16, 16)]
                plsc.addupdate_scatter(deg_v, [v], one16, mask=v < N)
        return 0

    lax.fori_loop(0, ROWS_PER_WORKER // BLK, _deg_blk, 0)

    nrows = (BP // CH) // (NC * NS)
    pltpu.sync_copy(batch_hbm.at[pl.ds(w * nrows, nrows)], blk_v)
    for r in range(nrows):
        for k in range(CH // 16):
            v = blk_v[r, pl.ds(k * 16, 16)]
            plsc.addupdate_scatter(cnt_v, [v], one16, mask=v < G)

    def _wr(i, _):
        pltpu.sync_copy(deg_v.at[pl.ds(i * 2000, 2000)],
                        degp_hbm.at[pl.ds(i * (NC * NS * 2000) + w * 2000,
                                          2000)])
        return 0

    lax.fori_loop(0, N // 2000, _wr, 0)
    pltpu.sync_copy(cnt_v, cntp_hbm.at[pl.ds(w * G, G)])


@functools.partial(
    pl.kernel,
    out_type=jax.ShapeDtypeStruct((NC * NP, 8), _f32),
    mesh=_mesh,
    compiler_params=_SC_PARAMS,
    scratch_types=[
        pltpu.VMEM_SHARED((NP, 8), _f32),
        pltpu.VMEM((CH, 8), _f32),
        pltpu.VMEM((BLK, CH), jnp.int32),
        pltpu.VMEM((BLK, CH), jnp.int32),
        pltpu.VMEM((2, CH, 8), _f32),
        pltpu.SemaphoreType.DMA,
        pltpu.SemaphoreType.DMA,
    ],
)
def _agg8(zeros_hbm, src_hbm, dst_hbm, t0_hbm, out_hbm, acc_sh, zb, sb, db,
          rb, sem0, sem1):
    c = lax.axis_index("c")
    s = lax.axis_index("s")
    w = s * NC + c
    sems = (sem0, sem1)
    pltpu.sync_copy(zeros_hbm, zb)

    def _zc(i, _):
        pltpu.sync_copy(zb, acc_sh.at[pl.ds(s * STRIPE + i * CH, CH)])
        return 0

    lax.fori_loop(0, STRIPE // CH, _zc, 0)
    plsc.subcore_barrier()

    def _blk(b, _):
        row0 = w * ROWS_PER_WORKER + b * BLK
        pltpu.sync_copy(src_hbm.at[pl.ds(row0, BLK)], sb)
        pltpu.sync_copy(dst_hbm.at[pl.ds(row0, BLK)], db)
        descs = [None, None]
        descs[0] = pltpu.async_copy(t0_hbm.at[sb.at[0]], rb.at[0], sems[0])
        for j in range(BLK):
            p = j % 2
            if j + 1 < BLK:
                q = (j + 1) % 2
                descs[q] = pltpu.async_copy(t0_hbm.at[sb.at[j + 1]], rb.at[q],
                                            sems[q])
            descs[p].wait()
            pltpu.sync_copy(rb.at[p], acc_sh.at[db.at[j]], add=True)
        return 0

    lax.fori_loop(0, ROWS_PER_WORKER // BLK, _blk, 0)
    plsc.subcore_barrier()
    pltpu.sync_copy(acc_sh.at[pl.ds(s * STRIPE, STRIPE)],
                    out_hbm.at[pl.ds(c * NP + s * STRIPE, STRIPE)])


def _agg32_body(src_hbm, dst_hbm, q_hbm, acc_sh, zb, sb, db, ib, rb, sems):
    c = lax.axis_index("c")
    s = lax.axis_index("s")

    def _zc(i, _):
        pltpu.sync_copy(zb, acc_sh.at[pl.ds(s * STRIPE + i * CH, CH)])
        return 0

    lax.fori_loop(0, STRIPE // CH, _zc, 0)
    plsc.subcore_barrier()

    coff = c * N

    def _blk(b, _):
        row0 = s * ROWS_PER_SC_TILE + b * BLK
        pltpu.sync_copy(src_hbm.at[pl.ds(row0, BLK)], sb)
        pltpu.sync_copy(dst_hbm.at[pl.ds(row0, BLK)], db)

        def _mkidx(j):
            p = j % 2
            for k in range(CH // 16):
                ib[p, pl.ds(k * 16, 16)] = sb[j, pl.ds(k * 16, 16)] + coff

        descs = [None, None]
        _mkidx(0)
        descs[0] = pltpu.async_copy(q_hbm.at[ib.at[0]], rb.at[0], sems[0])
        for j in range(BLK):
            p = j % 2
            if j + 1 < BLK:
                q = (j + 1) % 2
                _mkidx(j + 1)
                descs[q] = pltpu.async_copy(q_hbm.at[ib.at[q]], rb.at[q],
                                            sems[q])
            descs[p].wait()
            pltpu.sync_copy(rb.at[p], acc_sh.at[db.at[j]], add=True)
        return 0

    lax.fori_loop(0, ROWS_PER_SC_TILE // BLK, _blk, 0)


_AGG32_SCRATCH = [
    pltpu.VMEM_SHARED((NP, 32), _f32),
    pltpu.VMEM((CH, 32), _f32),
    pltpu.VMEM((BLK, CH), jnp.int32),
    pltpu.VMEM((BLK, CH), jnp.int32),
    pltpu.VMEM((2, CH), jnp.int32),
    pltpu.VMEM((2, CH, 32), _f32),
    pltpu.SemaphoreType.DMA,
    pltpu.SemaphoreType.DMA,
]


@functools.partial(
    pl.kernel,
    out_type=jax.ShapeDtypeStruct((NC * NP, 32), _f32),
    mesh=_mesh,
    compiler_params=_SC_PARAMS,
    scratch_types=_AGG32_SCRATCH,
)
def _agg32(zeros_hbm, src_hbm, dst_hbm, q_hbm, out_hbm, acc_sh, zb, sb, db,
           ib, rb, sem0, sem1):
    c = lax.axis_index("c")
    s = lax.axis_index("s")
    pltpu.sync_copy(zeros_hbm, zb)
    _agg32_body(src_hbm, dst_hbm, q_hbm, acc_sh, zb, sb, db, ib, rb,
                (sem0, sem1))
    plsc.subcore_barrier()
    pltpu.sync_copy(acc_sh.at[pl.ds(s * STRIPE, STRIPE)],
                    out_hbm.at[pl.ds(c * NP + s * STRIPE, STRIPE)])


@functools.partial(
    pl.kernel,
    out_type=jax.ShapeDtypeStruct((NC * G, 32), _f32),
    mesh=_mesh,
    compiler_params=_SC_PARAMS,
    scratch_types=_AGG32_SCRATCH + [
        pltpu.VMEM_SHARED((GP, 32), _f32),
        pltpu.VMEM((STRIPE,), _f32),
        pltpu.VMEM((STRIPE,), jnp.int32),
        pltpu.VMEM((CH,), jnp.int32),
        pltpu.VMEM((CH, 32), _f32),
    ],
)
def _agg32_pool(zeros_hbm, src_hbm, dst_hbm, q_hbm, dis_hbm, batch_hbm,
                out_hbm, acc_sh, zb, sb, db, ib, rb, sem0, sem1, pool_sh,
                dbuf, bbuf, bidx, pz):
    c = lax.axis_index("c")
    s = lax.axis_index("s")
    gstripe = GP // NS
    pltpu.sync_copy(zeros_hbm, zb)
    pltpu.sync_copy(zb.at[pl.ds(0, gstripe)],
                    pool_sh.at[pl.ds(s * gstripe, gstripe)])
    _agg32_body(src_hbm, dst_hbm, q_hbm, acc_sh, zb, sb, db, ib, rb,
                (sem0, sem1))
    plsc.subcore_barrier()

    pltpu.sync_copy(dis_hbm.at[pl.ds(s * STRIPE, STRIPE)], dbuf)
    pltpu.sync_copy(batch_hbm.at[pl.ds(s * STRIPE, STRIPE)], bbuf)

    def _pchunk(t, _):
        node0 = s * STRIPE + t * CH
        pltpu.sync_copy(acc_sh.at[pl.ds(node0, CH)], pz)
        for k in range(CH // 16):
            bidx[pl.ds(k * 16, 16)] = bbuf[pl.ds(t * CH + k * 16, 16)]

        def _srow(j, _):
            idx16 = lax.broadcast(t * CH + j, (16,))
            dsp = plsc.load_gather(dbuf, [idx16])
            pz[j, pl.ds(0, 16)] = pz[j, pl.ds(0, 16)] * dsp
            pz[j, pl.ds(16, 16)] = pz[j, pl.ds(16, 16)] * dsp
            return 0

        lax.fori_loop(0, CH, _srow, 0)
        pltpu.sync_copy(pz, pool_sh.at[bidx], add=True)
        return 0

    lax.fori_loop(0, STRIPE // CH, _pchunk, 0)
    plsc.subcore_barrier()
    gd = G // NS
    pltpu.sync_copy(pool_sh.at[pl.ds(s * gd, gd)],
                    out_hbm.at[pl.ds(c * G + s * gd, gd)])


_NB = 2000
_NBLK = N // _NB


def _tc0_body(degp_ref, x_ref, dis_ref, t0_ref):
    deg = jnp.sum(degp_ref[0], axis=0)
    dis = lax.rsqrt(deg)
    dis_ref[...] = dis[:, None]
    t0 = x_ref[...] * dis[:, None]
    t0_ref[...] = jnp.concatenate([t0, jnp.zeros((_NB, 1), _f32)], axis=1)


def _tc0(degp, x):
    return pl.pallas_call(
        _tc0_body,
        grid=(_NBLK,),
        in_specs=[
            pl.BlockSpec((1, NC * NS, _NB), lambda i: (i, 0, 0)),
            pl.BlockSpec((_NB, F_IN), lambda i: (i, 0)),
        ],
        out_specs=[
            pl.BlockSpec((_NB, 1), lambda i: (i, 0)),
            pl.BlockSpec((_NB, 8), lambda i: (i, 0)),
        ],
        out_shape=[
            jax.ShapeDtypeStruct((N, 1), _f32),
            jax.ShapeDtypeStruct((N, 8), _f32),
        ],
    )(degp, x)


def _tc_layer_body(split_in, ap_ref, dis_ref, w_ref, b_ref, q_ref):
    if split_in:
        a = jnp.concatenate([ap_ref[0], ap_ref[1]], axis=1)
    else:
        a = ap_ref[0] + ap_ref[1]
    dis = dis_ref[...]
    z = a * dis
    h = jnp.dot(z, w_ref[...], preferred_element_type=_f32)
    h = jnp.maximum(h + b_ref[...], 0.0)
    q = h * dis
    q_ref[0] = q[:, :32]
    q_ref[1] = q[:, 32:]


def _tc_layer(ap, dis, w, b, split_in):
    kin = ap.shape[-1]
    return pl.pallas_call(
        functools.partial(_tc_layer_body, split_in),
        grid=(_NBLK,),
        in_specs=[
            pl.BlockSpec((2, _NB, kin), lambda i: (0, i, 0)),
            pl.BlockSpec((_NB, 1), lambda i: (i, 0)),
            pl.BlockSpec(w.shape, lambda i: (0, 0)),
            pl.BlockSpec((1, H), lambda i: (0, 0)),
        ],
        out_specs=pl.BlockSpec((2, _NB, 32), lambda i: (0, i, 0)),
        out_shape=jax.ShapeDtypeStruct((2, N, 32), _f32),
    )(ap, dis, w, b)


def _tc_final_body(p_ref, cntp_ref, w3_ref, b3_ref, wl_ref, bl_ref, out_ref):
    counts = jnp.sum(cntp_ref[...], axis=0)
    p64 = jnp.concatenate([p_ref[0], p_ref[1]], axis=1)
    hs = jnp.dot(p64, w3_ref[...], preferred_element_type=_f32)
    hs = hs + counts[:, None] * b3_ref[...]
    pooled = hs / jnp.maximum(counts, 1.0)[:, None]
    out = jnp.dot(pooled, wl_ref[...], preferred_element_type=_f32)
    out_ref[...] = out + bl_ref[...]


def _tc_final(p, cntp, w3, b3, wl, bl):
    return pl.pallas_call(
        _tc_final_body,
        out_shape=jax.ShapeDtypeStruct((G, C), _f32),
    )(p, cntp, w3, b3, wl, bl)


def kernel(x, edge_index, batch, W1, b1, W2, b2, W3, b3, Wlin, blin):
    loop = jnp.arange(N, dtype=jnp.int32)
    pad = EP - (E + N)
    src = jnp.concatenate(
        [edge_index[0], loop,
         jnp.zeros((pad,), jnp.int32)]).reshape(NCHUNK, CH)
    dst = jnp.concatenate(
        [edge_index[1], loop,
         jnp.full((pad,), N, jnp.int32)]).reshape(NCHUNK, CH)
    batch_cnt = jnp.concatenate(
        [batch, jnp.full((BP - N,), G, jnp.int32)]).reshape(BP // CH, CH)
    batch_np = jnp.concatenate([batch, jnp.full((NP - N,), G, jnp.int32)])
    W1p = jnp.concatenate([W1, jnp.zeros((1, H), _f32)], axis=0)
    z8 = jnp.zeros((CH, 8), _f32)
    z32 = jnp.zeros((CH, 32), _f32)

    degp, cntp = _deg_counts(dst, batch_cnt)
    dis, t0 = _tc0(degp.reshape(N // 2000, NC * NS, 2000), x)
    a1p = _agg8(z8, src, dst, t0)
    q1 = _tc_layer(a1p.reshape(2, NP, 8)[:, :N], dis, W1p,
                   b1.reshape(1, H), False)
    a2 = _agg32(z32, src, dst, q1.reshape(NC * N, 32))
    q2 = _tc_layer(a2.reshape(2, NP, 32)[:, :N], dis, W2,
                   b2.reshape(1, H), True)
    dis_np = jnp.concatenate(
        [dis.reshape(N), jnp.ones((NP - N,), _f32)])
    p = _agg32_pool(z32, src, dst, q2.reshape(NC * N, 32), dis_np, batch_np)
    return _tc_final(p.reshape(2, G, 32), cntp.reshape(NC * NS, G),
                     W3, b3.reshape(1, H), Wlin, blin.reshape(1, C))


# speedup vs baseline: 26.3867x; 1.0341x over previous
import functools

import jax
import jax.numpy as jnp
from jax import lax
from jax.experimental import pallas as pl
from jax.experimental.pallas import tpu as pltpu
from jax.experimental.pallas import tpu_sc as plsc

N = 50000
F_IN = 7
H = 64
C = 2
G = 512
E = 800000

NC = 2
NS = 16
CH = 128
EP = 851968
NCHUNK = EP // CH
ROWS_PER_SC_TILE = NCHUNK // NS
ROWS_PER_WORKER = NCHUNK // (NC * NS)
BLK = 16
NP = 51200
STRIPE = NP // NS
BP = 65536
GP = 640

_mesh = plsc.VectorSubcoreMesh(core_axis_name="c", subcore_axis_name="s")
_f32 = jnp.float32
_SC_PARAMS = pltpu.CompilerParams(needs_layout_passes=False,
                                  use_tc_tiling_on_sc=False)


@functools.partial(
    pl.kernel,
    out_type=(jax.ShapeDtypeStruct((NC * NS * N,), _f32),
              jax.ShapeDtypeStruct((NC * NS * G,), _f32)),
    mesh=_mesh,
    compiler_params=_SC_PARAMS,
    scratch_types=[
        pltpu.VMEM((N,), _f32),
        pltpu.VMEM((G,), _f32),
        pltpu.VMEM((BLK, CH), jnp.int32),
    ],
)
def _deg_counts(dst_hbm, batch_hbm, degp_hbm, cntp_hbm, deg_v, cnt_v, blk_v):
    c = lax.axis_index("c")
    s = lax.axis_index("s")
    w = s * NC + c
    zero16 = jnp.zeros((16,), _f32)
    one16 = jnp.ones((16,), _f32)

    def _z(i, _):
        deg_v[pl.ds(i * 16, 16)] = zero16
        return 0

    lax.fori_loop(0, N // 16, _z, 0)
    for i in range(G // 16):
        cnt_v[pl.ds(i * 16, 16)] = zero16

    def _deg_blk(b, _):
        pltpu.sync_copy(dst_hbm.at[pl.ds(w * ROWS_PER_WORKER + b * BLK, BLK)],
                        blk_v)
        for r in range(BLK):
            for k in range(CH // 16):
                v = blk_v[r, pl.ds(k * 16, 16)]
                plsc.addupdate_scatter(deg_v, [v], one16, mask=v < N)
        return 0

    lax.fori_loop(0, ROWS_PER_WORKER // BLK, _deg_blk, 0)

    nrows = (BP // CH) // (NC * NS)
    pltpu.sync_copy(batch_hbm.at[pl.ds(w * nrows, nrows)], blk_v)
    for r in range(nrows):
        for k in range(CH // 16):
            v = blk_v[r, pl.ds(k * 16, 16)]
            plsc.addupdate_scatter(cnt_v, [v], one16, mask=v < G)

    def _wr(i, _):
        pltpu.sync_copy(deg_v.at[pl.ds(i * 2000, 2000)],
                        degp_hbm.at[pl.ds(i * (NC * NS * 2000) + w * 2000,
                                          2000)])
        return 0

    lax.fori_loop(0, N // 2000, _wr, 0)
    pltpu.sync_copy(cnt_v, cntp_hbm.at[pl.ds(w * G, G)])


@functools.partial(
    pl.kernel,
    out_type=jax.ShapeDtypeStruct((NC * NP, 8), _f32),
    mesh=_mesh,
    compiler_params=_SC_PARAMS,
    scratch_types=[
        pltpu.VMEM_SHARED((NP, 8), _f32),
        pltpu.VMEM((BLK, CH), jnp.int32),
        pltpu.VMEM((BLK, CH), jnp.int32),
        pltpu.VMEM((4, CH, 8), _f32),
        pltpu.SemaphoreType.DMA,
        pltpu.SemaphoreType.DMA,
        pltpu.SemaphoreType.DMA,
        pltpu.SemaphoreType.DMA,
    ],
)
def _agg8(zeros_hbm, src_hbm, dst_hbm, t0_hbm, out_hbm, acc_sh, sb, db,
          rb, gsem0, gsem1, ssem0, ssem1):
    c = lax.axis_index("c")
    s = lax.axis_index("s")
    w = s * NC + c
    gsems = (gsem0, gsem1)
    ssems = (ssem0, ssem1)

    def _zc(i, _):
        pltpu.sync_copy(zeros_hbm, acc_sh.at[pl.ds(s * STRIPE + i * CH, CH)])
        return 0

    lax.fori_loop(0, STRIPE // CH, _zc, 0)
    plsc.subcore_barrier()

    def _blk(b, _):
        row0 = w * ROWS_PER_WORKER + b * BLK
        pltpu.sync_copy(src_hbm.at[pl.ds(row0, BLK)], sb)
        pltpu.sync_copy(dst_hbm.at[pl.ds(row0, BLK)], db)
        gd = [None] * 4
        sd = [None] * BLK
        gd[0] = pltpu.async_copy(t0_hbm.at[sb.at[0]], rb.at[0], gsems[0])
        gd[1] = pltpu.async_copy(t0_hbm.at[sb.at[1]], rb.at[1], gsems[1])
        for j in range(BLK):
            gd[j % 4].wait()
            sd[j] = pltpu.async_copy(rb.at[j % 4], acc_sh.at[db.at[j]],
                                     ssems[j % 2], add=True)
            nj = j + 2
            if nj < BLK:
                if nj >= 4:
                    sd[nj - 4].wait()
                gd[nj % 4] = pltpu.async_copy(t0_hbm.at[sb.at[nj]],
                                              rb.at[nj % 4], gsems[nj % 2])
        for j in range(BLK - 4, BLK):
            sd[j].wait()
        return 0

    lax.fori_loop(0, ROWS_PER_WORKER // BLK, _blk, 0)
    plsc.subcore_barrier()
    pltpu.sync_copy(acc_sh.at[pl.ds(s * STRIPE, STRIPE)],
                    out_hbm.at[pl.ds(c * NP + s * STRIPE, STRIPE)])


def _agg32_body(zeros_hbm, src_hbm, dst_hbm, q_hbm, acc_sh, sb, db, ib, rb,
                gsems, ssems):
    c = lax.axis_index("c")
    s = lax.axis_index("s")

    def _zc(i, _):
        pltpu.sync_copy(zeros_hbm, acc_sh.at[pl.ds(s * STRIPE + i * CH, CH)])
        return 0

    lax.fori_loop(0, STRIPE // CH, _zc, 0)
    plsc.subcore_barrier()

    coff = c * N

    def _blk(b, _):
        row0 = s * ROWS_PER_SC_TILE + b * BLK
        pltpu.sync_copy(src_hbm.at[pl.ds(row0, BLK)], sb)
        pltpu.sync_copy(dst_hbm.at[pl.ds(row0, BLK)], db)

        def _mkidx(j):
            pp = j % 2
            for k in range(CH // 16):
                ib[pp, pl.ds(k * 16, 16)] = sb[j, pl.ds(k * 16, 16)] + coff

        gd = [None] * 4
        sd = [None] * BLK
        _mkidx(0)
        gd[0] = pltpu.async_copy(q_hbm.at[ib.at[0]], rb.at[0], gsems[0])
        _mkidx(1)
        gd[1] = pltpu.async_copy(q_hbm.at[ib.at[1]], rb.at[1], gsems[1])
        for j in range(BLK):
            gd[j % 4].wait()
            sd[j] = pltpu.async_copy(rb.at[j % 4], acc_sh.at[db.at[j]],
                                     ssems[j % 2], add=True)
            nj = j + 2
            if nj < BLK:
                if nj >= 4:
                    sd[nj - 4].wait()
                _mkidx(nj)
                gd[nj % 4] = pltpu.async_copy(q_hbm.at[ib.at[nj % 2]],
                                              rb.at[nj % 4], gsems[nj % 2])
        for j in range(BLK - 4, BLK):
            sd[j].wait()
        return 0

    lax.fori_loop(0, ROWS_PER_SC_TILE // BLK, _blk, 0)


_AGG32_SCRATCH = [
    pltpu.VMEM_SHARED((NP, 32), _f32),
    pltpu.VMEM((BLK, CH), jnp.int32),
    pltpu.VMEM((BLK, CH), jnp.int32),
    pltpu.VMEM((2, CH), jnp.int32),
    pltpu.VMEM((4, CH, 32), _f32),
    pltpu.SemaphoreType.DMA,
    pltpu.SemaphoreType.DMA,
    pltpu.SemaphoreType.DMA,
    pltpu.SemaphoreType.DMA,
]


@functools.partial(
    pl.kernel,
    out_type=jax.ShapeDtypeStruct((NC * NP, 32), _f32),
    mesh=_mesh,
    compiler_params=_SC_PARAMS,
    scratch_types=_AGG32_SCRATCH,
)
def _agg32(zeros_hbm, src_hbm, dst_hbm, q_hbm, out_hbm, acc_sh, sb, db,
           ib, rb, gsem0, gsem1, ssem0, ssem1):
    c = lax.axis_index("c")
    s = lax.axis_index("s")
    _agg32_body(zeros_hbm, src_hbm, dst_hbm, q_hbm, acc_sh, sb, db, ib, rb,
                (gsem0, gsem1), (ssem0, ssem1))
    plsc.subcore_barrier()
    pltpu.sync_copy(acc_sh.at[pl.ds(s * STRIPE, STRIPE)],
                    out_hbm.at[pl.ds(c * NP + s * STRIPE, STRIPE)])


@functools.partial(
    pl.kernel,
    out_type=jax.ShapeDtypeStruct((NC * G, 32), _f32),
    mesh=_mesh,
    compiler_params=_SC_PARAMS,
    scratch_types=_AGG32_SCRATCH + [
        pltpu.VMEM_SHARED((GP, 32), _f32),
        pltpu.VMEM((CH,), _f32),
        pltpu.VMEM((CH,), jnp.int32),
        pltpu.VMEM((CH, 32), _f32),
    ],
)
def _agg32_pool(zeros_hbm, src_hbm, dst_hbm, q_hbm, dis_hbm, batch_hbm,
                out_hbm, acc_sh, sb, db, ib, rb, gsem0, gsem1, ssem0,
                ssem1, pool_sh, dbuf, bidx, pz):
    c = lax.axis_index("c")
    s = lax.axis_index("s")
    gstripe = GP // NS
    pltpu.sync_copy(zeros_hbm.at[pl.ds(0, gstripe)],
                    pool_sh.at[pl.ds(s * gstripe, gstripe)])
    _agg32_body(zeros_hbm, src_hbm, dst_hbm, q_hbm, acc_sh, sb, db, ib, rb,
                (gsem0, gsem1), (ssem0, ssem1))
    plsc.subcore_barrier()

    def _pchunk(t, _):
        node0 = s * STRIPE + t * CH
        pltpu.sync_copy(acc_sh.at[pl.ds(node0, CH)], pz)
        pltpu.sync_copy(dis_hbm.at[pl.ds(node0, CH)], dbuf)
        pltpu.sync_copy(batch_hbm.at[pl.ds(node0, CH)], bidx)

        def _srow(j, _):
            idx16 = lax.broadcast(j, (16,))
            dsp = plsc.load_gather(dbuf, [idx16])
            pz[j, pl.ds(0, 16)] = pz[j, pl.ds(0, 16)] * dsp
            pz[j, pl.ds(16, 16)] = pz[j, pl.ds(16, 16)] * dsp
            return 0

        lax.fori_loop(0, CH, _srow, 0)
        pltpu.sync_copy(pz, pool_sh.at[bidx], add=True)
        return 0

    lax.fori_loop(0, STRIPE // CH, _pchunk, 0)
    plsc.subcore_barrier()
    gd = G // NS
    pltpu.sync_copy(pool_sh.at[pl.ds(s * gd, gd)],
                    out_hbm.at[pl.ds(c * G + s * gd, gd)])


_NB = 2000
_NBLK = N // _NB


def _tc0_body(degp_ref, x_ref, dis_ref, t0_ref):
    deg = jnp.sum(degp_ref[0], axis=0)
    dis = lax.rsqrt(deg)
    dis_ref[...] = dis[:, None]
    t0 = x_ref[...] * dis[:, None]
    t0_ref[...] = jnp.concatenate([t0, jnp.zeros((_NB, 1), _f32)], axis=1)


def _tc0(degp, x):
    return pl.pallas_call(
        _tc0_body,
        grid=(_NBLK,),
        in_specs=[
            pl.BlockSpec((1, NC * NS, _NB), lambda i: (i, 0, 0)),
            pl.BlockSpec((_NB, F_IN), lambda i: (i, 0)),
        ],
        out_specs=[
            pl.BlockSpec((_NB, 1), lambda i: (i, 0)),
            pl.BlockSpec((_NB, 8), lambda i: (i, 0)),
        ],
        out_shape=[
            jax.ShapeDtypeStruct((N, 1), _f32),
            jax.ShapeDtypeStruct((N, 8), _f32),
        ],
    )(degp, x)


def _tc_layer_body(split_in, ap_ref, dis_ref, w_ref, b_ref, q_ref):
    if split_in:
        a = jnp.concatenate([ap_ref[0], ap_ref[1]], axis=1)
    else:
        a = ap_ref[0] + ap_ref[1]
    dis = dis_ref[...]
    z = a * dis
    h = jnp.dot(z, w_ref[...], preferred_element_type=_f32)
    h = jnp.maximum(h + b_ref[...], 0.0)
    q = h * dis
    q_ref[0] = q[:, :32]
    q_ref[1] = q[:, 32:]


def _tc_layer(ap, dis, w, b, split_in):
    kin = ap.shape[-1]
    return pl.pallas_call(
        functools.partial(_tc_layer_body, split_in),
        grid=(_NBLK,),
        in_specs=[
            pl.BlockSpec((2, _NB, kin), lambda i: (0, i, 0)),
            pl.BlockSpec((_NB, 1), lambda i: (i, 0)),
            pl.BlockSpec(w.shape, lambda i: (0, 0)),
            pl.BlockSpec((1, H), lambda i: (0, 0)),
        ],
        out_specs=pl.BlockSpec((2, _NB, 32), lambda i: (0, i, 0)),
        out_shape=jax.ShapeDtypeStruct((2, N, 32), _f32),
    )(ap, dis, w, b)


def _tc_final_body(p_ref, cntp_ref, w3_ref, b3_ref, wl_ref, bl_ref, out_ref):
    counts = jnp.sum(cntp_ref[...], axis=0)
    p64 = jnp.concatenate([p_ref[0], p_ref[1]], axis=1)
    hs = jnp.dot(p64, w3_ref[...], preferred_element_type=_f32)
    hs = hs + counts[:, None] * b3_ref[...]
    pooled = hs / jnp.maximum(counts, 1.0)[:, None]
    out = jnp.dot(pooled, wl_ref[...], preferred_element_type=_f32)
    out_ref[...] = out + bl_ref[...]


def _tc_final(p, cntp, w3, b3, wl, bl):
    return pl.pallas_call(
        _tc_final_body,
        out_shape=jax.ShapeDtypeStruct((G, C), _f32),
    )(p, cntp, w3, b3, wl, bl)


def kernel(x, edge_index, batch, W1, b1, W2, b2, W3, b3, Wlin, blin):
    loop = jnp.arange(N, dtype=jnp.int32)
    pad = EP - (E + N)
    src = jnp.concatenate(
        [edge_index[0], loop,
         jnp.zeros((pad,), jnp.int32)]).reshape(NCHUNK, CH)
    dst = jnp.concatenate(
        [edge_index[1], loop,
         jnp.full((pad,), N, jnp.int32)]).reshape(NCHUNK, CH)
    batch_cnt = jnp.concatenate(
        [batch, jnp.full((BP - N,), G, jnp.int32)]).reshape(BP // CH, CH)
    batch_np = jnp.concatenate([batch, jnp.full((NP - N,), G, jnp.int32)])
    W1p = jnp.concatenate([W1, jnp.zeros((1, H), _f32)], axis=0)
    z8 = jnp.zeros((CH, 8), _f32)
    z32 = jnp.zeros((CH, 32), _f32)

    degp, cntp = _deg_counts(dst, batch_cnt)
    dis, t0 = _tc0(degp.reshape(N // 2000, NC * NS, 2000), x)
    a1p = _agg8(z8, src, dst, t0)
    q1 = _tc_layer(a1p.reshape(2, NP, 8), dis, W1p, b1.reshape(1, H), False)
    a2 = _agg32(z32, src, dst, q1.reshape(NC * N, 32))
    q2 = _tc_layer(a2.reshape(2, NP, 32), dis, W2, b2.reshape(1, H), True)
    dis_np = jnp.concatenate(
        [dis.reshape(N), jnp.ones((NP - N,), _f32)])
    p = _agg32_pool(z32, src, dst, q2.reshape(NC * N, 32), dis_np, batch_np)
    return _tc_final(p.reshape(2, G, 32), cntp.reshape(NC * NS, G),
                     W3, b3.reshape(1, H), Wlin, blin.reshape(1, C))
